# baseline (speedup 1.0000x reference)
import jax
import jax.numpy as jnp
from jax import lax
from jax.experimental import pallas as pl
from jax.experimental.pallas import tpu as pltpu

_VMEM_LIMIT = 48 * 1024 * 1024


def _pick_tile(n: int, candidates) -> int:
    for c in candidates:
        if c <= n and n % c == 0:
            return c
    return n


def _prep_kernel(ci_ref, cb_ref, n2_ref):
    c = ci_ref[...].astype(jnp.float32)
    cb_ref[...] = c.astype(jnp.bfloat16)
    n2_ref[...] = jnp.sum(c * c, axis=1, keepdims=True)


def _match_kernel(ct_ref, call_ref, n2c_ref, n2r_ref, first_ref):
    tq = first_ref.shape[0]
    n = call_ref.shape[0]
    g = lax.dot_general(ct_ref[...], call_ref[...],
                        dimension_numbers=(((1,), (1,)), ((), ())),
                        preferred_element_type=jnp.float32)
    d2 = (n2c_ref[...] + n2r_ref[...]) - (g + g)
    cj = lax.broadcasted_iota(jnp.int32, (tq, n), 1)
    first_ref[...] = jnp.min(jnp.where(d2 > 0.5, n, cj),
                             axis=1, keepdims=True)


def _colors_kernel(fc_ref, fr_ref, out_ref):
    tq = out_ref.shape[0]
    n = fr_ref.shape[1]
    rep = fr_ref[...] == lax.broadcasted_iota(jnp.int32, (1, n), 1)
    cj = lax.broadcasted_iota(jnp.int32, (tq, n), 1)
    counted = jnp.logical_and(rep, cj < fc_ref[...])
    out_ref[...] = jnp.sum(counted.astype(jnp.int32), axis=1, keepdims=True)


def kernel(x_labels, edge_index):
    N = int(x_labels.shape[0])
    C = 64
    Cp = 128
    src, dst = edge_index[0], edge_index[1]
    x32 = x_labels.astype(jnp.int32)

    e_idx = dst * Cp + x32[src]
    s_idx = jnp.arange(N, dtype=jnp.int32) * Cp + (C + x32)
    flat = jnp.zeros((N * Cp,), jnp.int32)
    flat = flat.at[jnp.concatenate([e_idx, s_idx])].add(
        1, mode="drop", unique_indices=False, indices_are_sorted=False)
    sig_i32 = flat.reshape(N, Cp)

    tp = _pick_tile(N, (1024, 512, 256, 128, 64, 32, 16, 8))
    sig_bf16, n2 = pl.pallas_call(
        _prep_kernel,
        out_shape=(jax.ShapeDtypeStruct((N, Cp), jnp.bfloat16),
                   jax.ShapeDtypeStruct((N, 1), jnp.float32)),
        grid=(N // tp,),
        in_specs=[pl.BlockSpec((tp, Cp), lambda i: (i, 0))],
        out_specs=(pl.BlockSpec((tp, Cp), lambda i: (i, 0)),
                   pl.BlockSpec((tp, 1), lambda i: (i, 0))),
        compiler_params=pltpu.CompilerParams(
            dimension_semantics=("parallel",),
            vmem_limit_bytes=_VMEM_LIMIT),
    )(sig_i32)

    tq = _pick_tile(N, (256, 128, 64, 32, 16, 8))
    first = pl.pallas_call(
        _match_kernel,
        out_shape=jax.ShapeDtypeStruct((N, 1), jnp.int32),
        grid=(N // tq,),
        in_specs=[
            pl.BlockSpec((tq, Cp), lambda i: (i, 0)),
            pl.BlockSpec((N, Cp), lambda i: (0, 0)),
            pl.BlockSpec((tq, 1), lambda i: (i, 0)),
            pl.BlockSpec((1, N), lambda i: (0, 0)),
        ],
        out_specs=pl.BlockSpec((tq, 1), lambda i: (i, 0)),
        compiler_params=pltpu.CompilerParams(
            dimension_semantics=("parallel",),
            vmem_limit_bytes=_VMEM_LIMIT),
    )(sig_bf16, sig_bf16, n2, n2.reshape(1, N))

    colors = pl.pallas_call(
        _colors_kernel,
        out_shape=jax.ShapeDtypeStruct((N, 1), jnp.int32),
        grid=(N // tq,),
        in_specs=[
            pl.BlockSpec((tq, 1), lambda i: (i, 0)),
            pl.BlockSpec((1, N), lambda i: (0, 0)),
        ],
        out_specs=pl.BlockSpec((tq, 1), lambda i: (i, 0)),
        compiler_params=pltpu.CompilerParams(
            dimension_semantics=("parallel",),
            vmem_limit_bytes=_VMEM_LIMIT),
    )(first, first.reshape(1, N))

    return colors[:, 0]


# speedup vs baseline: 18.6316x; 18.6316x over previous
import jax
import jax.numpy as jnp
from jax import lax
from jax.experimental import pallas as pl
from jax.experimental.pallas import tpu as pltpu

_VMEM_LIMIT = 48 * 1024 * 1024


def _pick_tile(n: int, candidates) -> int:
    for c in candidates:
        if c <= n and n % c == 0:
            return c
    return n


def _edge_idx_kernel(src_ref, dst_ref, x2t_ref, idx_ref):
    r, l = src_ref.shape[1], src_ref.shape[2]
    s = src_ref[0]
    lo = s & 127
    hi = s >> 7
    oh_lo = (lo[:, :, None] ==
             lax.broadcasted_iota(jnp.int32, (r, l, 128), 2)
             ).astype(jnp.bfloat16).reshape(r * l, 128)
    y = jnp.dot(oh_lo, x2t_ref[...],
                preferred_element_type=jnp.float32)
    oh_hi = (hi[:, :, None] ==
             lax.broadcasted_iota(jnp.int32, (r, l, 64), 2)
             ).astype(jnp.float32).reshape(r * l, 64)
    lab = jnp.sum(y * oh_hi, axis=1).reshape(r, l)
    idx_ref[0] = dst_ref[0] * 128 + lab.astype(jnp.int32)


def _prep_kernel(ci_ref, cb_ref, n2_ref):
    c = ci_ref[...].astype(jnp.float32)
    cb_ref[...] = c.astype(jnp.bfloat16)
    n2_ref[...] = jnp.sum(c * c, axis=1, keepdims=True)


def _match_kernel(ct_ref, call_ref, n2c_ref, n2r_ref, first_ref):
    tq = first_ref.shape[0]
    n = call_ref.shape[0]
    g = lax.dot_general(ct_ref[...], call_ref[...],
                        dimension_numbers=(((1,), (1,)), ((), ())),
                        preferred_element_type=jnp.float32)
    d2 = (n2c_ref[...] + n2r_ref[...]) - (g + g)
    cj = lax.broadcasted_iota(jnp.int32, (tq, n), 1)
    first_ref[...] = jnp.min(jnp.where(d2 > 0.5, n, cj),
                             axis=1, keepdims=True)


def _colors_kernel(fc_ref, fr_ref, out_ref):
    tq = out_ref.shape[0]
    n = fr_ref.shape[1]
    rep = fr_ref[...] == lax.broadcasted_iota(jnp.int32, (1, n), 1)
    cj = lax.broadcasted_iota(jnp.int32, (tq, n), 1)
    counted = jnp.logical_and(rep, cj < fc_ref[...])
    out_ref[...] = jnp.sum(counted.astype(jnp.int32), axis=1, keepdims=True)


def kernel(x_labels, edge_index):
    N = int(x_labels.shape[0])
    E = int(edge_index.shape[1])
    C = 64
    Cp = 128
    src, dst = edge_index[0], edge_index[1]
    x32 = x_labels.astype(jnp.int32)

    eb = 32768
    while E % eb:
        eb //= 2
    rr = eb // 128
    g = E // eb
    src3 = src.reshape(g, rr, 128)
    dst3 = dst.reshape(g, rr, 128)
    x2t = x32.reshape(C, Cp).T.astype(jnp.bfloat16)

    e_idx = pl.pallas_call(
        _edge_idx_kernel,
        out_shape=jax.ShapeDtypeStruct((g, rr, 128), jnp.int32),
        grid=(g,),
        in_specs=[
            pl.BlockSpec((1, rr, 128), lambda i: (i, 0, 0)),
            pl.BlockSpec((1, rr, 128), lambda i: (i, 0, 0)),
            pl.BlockSpec((Cp, C), lambda i: (0, 0)),
        ],
        out_specs=pl.BlockSpec((1, rr, 128), lambda i: (i, 0, 0)),
        compiler_params=pltpu.CompilerParams(
            dimension_semantics=("parallel",),
            vmem_limit_bytes=_VMEM_LIMIT),
    )(src3, dst3, x2t)

    s_idx = jnp.arange(N, dtype=jnp.int32) * Cp + (C + x32)
    flat = jnp.zeros((N * Cp,), jnp.int32)
    flat = flat.at[jnp.concatenate([e_idx.reshape(E), s_idx])].add(1)
    sig_i32 = flat.reshape(N, Cp)

    tp = _pick_tile(N, (1024, 512, 256, 128, 64, 32, 16, 8))
    sig_bf16, n2 = pl.pallas_call(
        _prep_kernel,
        out_shape=(jax.ShapeDtypeStruct((N, Cp), jnp.bfloat16),
                   jax.ShapeDtypeStruct((N, 1), jnp.float32)),
        grid=(N // tp,),
        in_specs=[pl.BlockSpec((tp, Cp), lambda i: (i, 0))],
        out_specs=(pl.BlockSpec((tp, Cp), lambda i: (i, 0)),
                   pl.BlockSpec((tp, 1), lambda i: (i, 0))),
        compiler_params=pltpu.CompilerParams(
            dimension_semantics=("parallel",),
            vmem_limit_bytes=_VMEM_LIMIT),
    )(sig_i32)

    tq = _pick_tile(N, (256, 128, 64, 32, 16, 8))
    first = pl.pallas_call(
        _match_kernel,
        out_shape=jax.ShapeDtypeStruct((N, 1), jnp.int32),
        grid=(N // tq,),
        in_specs=[
            pl.BlockSpec((tq, Cp), lambda i: (i, 0)),
            pl.BlockSpec((N, Cp), lambda i: (0, 0)),
            pl.BlockSpec((tq, 1), lambda i: (i, 0)),
            pl.BlockSpec((1, N), lambda i: (0, 0)),
        ],
        out_specs=pl.BlockSpec((tq, 1), lambda i: (i, 0)),
        compiler_params=pltpu.CompilerParams(
            dimension_semantics=("parallel",),
            vmem_limit_bytes=_VMEM_LIMIT),
    )(sig_bf16, sig_bf16, n2, n2.reshape(1, N))

    colors = pl.pallas_call(
        _colors_kernel,
        out_shape=jax.ShapeDtypeStruct((N, 1), jnp.int32),
        grid=(N // tq,),
        in_specs=[
            pl.BlockSpec((tq, 1), lambda i: (i, 0)),
            pl.BlockSpec((1, N), lambda i: (0, 0)),
        ],
        out_specs=pl.BlockSpec((tq, 1), lambda i: (i, 0)),
        compiler_params=pltpu.CompilerParams(
            dimension_semantics=("parallel",),
            vmem_limit_bytes=_VMEM_LIMIT),
    )(first, first.reshape(1, N))

    return colors[:, 0]


# speedup vs baseline: 49.7505x; 2.6702x over previous
import jax
import jax.numpy as jnp
from jax import lax
from jax.experimental import pallas as pl
from jax.experimental.pallas import tpu as pltpu

_VMEM_LIMIT = 48 * 1024 * 1024


def _pick_tile(n: int, candidates) -> int:
    for c in candidates:
        if c <= n and n % c == 0:
            return c
    return n


def _edge_idx_kernel(src_ref, dst_ref, x2_ref, idx_ref):
    b = src_ref.shape[2]
    s = src_ref[0]
    lo = s & 127
    hi = s >> 7
    oh_lo = (lax.broadcasted_iota(jnp.int32, (128, b), 0) == lo
             ).astype(jnp.bfloat16)
    y = jnp.dot(x2_ref[...], oh_lo,
                preferred_element_type=jnp.float32)
    hi_eq = lax.broadcasted_iota(jnp.int32, (64, b), 0) == hi
    lab = jnp.sum(jnp.where(hi_eq, y, 0.0), axis=0, keepdims=True)
    idx_ref[0] = dst_ref[0] * 128 + lab.astype(jnp.int32)


def _prep_kernel(ci_ref, cb_ref, n2_ref):
    c = ci_ref[...].astype(jnp.float32)
    cb_ref[...] = c.astype(jnp.bfloat16)
    n2_ref[...] = jnp.sum(c * c, axis=1, keepdims=True)


def _match_kernel(ct_ref, call_ref, n2c_ref, n2r_ref, first_ref):
    tq = first_ref.shape[0]
    n = call_ref.shape[0]
    g = lax.dot_general(ct_ref[...], call_ref[...],
                        dimension_numbers=(((1,), (1,)), ((), ())),
                        preferred_element_type=jnp.float32)
    d2 = (n2c_ref[...] + n2r_ref[...]) - (g + g)
    cj = lax.broadcasted_iota(jnp.int32, (tq, n), 1)
    first_ref[...] = jnp.min(jnp.where(d2 > 0.5, n, cj),
                             axis=1, keepdims=True)


def _colors_kernel(fc_ref, fr_ref, out_ref):
    tq = out_ref.shape[0]
    n = fr_ref.shape[1]
    rep = fr_ref[...] == lax.broadcasted_iota(jnp.int32, (1, n), 1)
    cj = lax.broadcasted_iota(jnp.int32, (tq, n), 1)
    counted = jnp.logical_and(rep, cj < fc_ref[...])
    out_ref[...] = jnp.sum(counted.astype(jnp.int32), axis=1, keepdims=True)


def kernel(x_labels, edge_index):
    N = int(x_labels.shape[0])
    E = int(edge_index.shape[1])
    C = 64
    Cp = 128
    src, dst = edge_index[0], edge_index[1]
    x32 = x_labels.astype(jnp.int32)

    eb = 32768
    while E % eb:
        eb //= 2
    g = E // eb
    src3 = src.reshape(g, 1, eb)
    dst3 = dst.reshape(g, 1, eb)
    x2 = x32.reshape(C, Cp).astype(jnp.bfloat16)

    e_idx = pl.pallas_call(
        _edge_idx_kernel,
        out_shape=jax.ShapeDtypeStruct((g, 1, eb), jnp.int32),
        grid=(g,),
        in_specs=[
            pl.BlockSpec((1, 1, eb), lambda i: (i, 0, 0)),
            pl.BlockSpec((1, 1, eb), lambda i: (i, 0, 0)),
            pl.BlockSpec((C, Cp), lambda i: (0, 0)),
        ],
        out_specs=pl.BlockSpec((1, 1, eb), lambda i: (i, 0, 0)),
        compiler_params=pltpu.CompilerParams(
            dimension_semantics=("parallel",),
            vmem_limit_bytes=_VMEM_LIMIT),
    )(src3, dst3, x2)

    s_idx = jnp.arange(N, dtype=jnp.int32) * Cp + (C + x32)
    flat = jnp.zeros((N * Cp,), jnp.int32)
    flat = flat.at[jnp.concatenate([e_idx.reshape(E), s_idx])].add(1)
    sig_i32 = flat.reshape(N, Cp)

    tp = _pick_tile(N, (1024, 512, 256, 128, 64, 32, 16, 8))
    sig_bf16, n2 = pl.pallas_call(
        _prep_kernel,
        out_shape=(jax.ShapeDtypeStruct((N, Cp), jnp.bfloat16),
                   jax.ShapeDtypeStruct((N, 1), jnp.float32)),
        grid=(N // tp,),
        in_specs=[pl.BlockSpec((tp, Cp), lambda i: (i, 0))],
        out_specs=(pl.BlockSpec((tp, Cp), lambda i: (i, 0)),
                   pl.BlockSpec((tp, 1), lambda i: (i, 0))),
        compiler_params=pltpu.CompilerParams(
            dimension_semantics=("parallel",),
            vmem_limit_bytes=_VMEM_LIMIT),
    )(sig_i32)

    tq = _pick_tile(N, (256, 128, 64, 32, 16, 8))
    first = pl.pallas_call(
        _match_kernel,
        out_shape=jax.ShapeDtypeStruct((N, 1), jnp.int32),
        grid=(N // tq,),
        in_specs=[
            pl.BlockSpec((tq, Cp), lambda i: (i, 0)),
            pl.BlockSpec((N, Cp), lambda i: (0, 0)),
            pl.BlockSpec((tq, 1), lambda i: (i, 0)),
            pl.BlockSpec((1, N), lambda i: (0, 0)),
        ],
        out_specs=pl.BlockSpec((tq, 1), lambda i: (i, 0)),
        compiler_params=pltpu.CompilerParams(
            dimension_semantics=("parallel",),
            vmem_limit_bytes=_VMEM_LIMIT),
    )(sig_bf16, sig_bf16, n2, n2.reshape(1, N))

    colors = pl.pallas_call(
        _colors_kernel,
        out_shape=jax.ShapeDtypeStruct((N, 1), jnp.int32),
        grid=(N // tq,),
        in_specs=[
            pl.BlockSpec((tq, 1), lambda i: (i, 0)),
            pl.BlockSpec((1, N), lambda i: (0, 0)),
        ],
        out_specs=pl.BlockSpec((tq, 1), lambda i: (i, 0)),
        compiler_params=pltpu.CompilerParams(
            dimension_semantics=("parallel",),
            vmem_limit_bytes=_VMEM_LIMIT),
    )(first, first.reshape(1, N))

    return colors[:, 0]
